# Initial kernel scaffold; baseline (speedup 1.0000x reference)
import jax
import jax.numpy as jnp
from jax.experimental import pallas as pl


def kernel(feature, conv_w, conv_b, cls_w, cls_b, reg_w, reg_b):
    raise NotImplementedError("write your pallas kernel here")


# speedup vs baseline: 31.4499x; 31.4499x over previous
import math

import jax
import jax.numpy as jnp
import numpy as np
from jax.experimental import pallas as pl

_BASE_SIZE, _STRIDE = 16, 16
_SCALES = (8, 16, 32)
_RATIOS = (0.5, 1, 1.5)
_NA = len(_SCALES) * len(_RATIOS)
_IM_W, _IM_H = 1024, 1024
_MIN_SIZE, _SCALE_F = 16, 1.0
_NMS_THRESH = 0.7
_PRE_NMS, _POST_NMS = 2000, 2000
_H, _W = 64, 64
_N = _H * _W * _NA
_NR, _NC = 288, 128
_PAD_N = 2048


def _anchor_params():
    cx = cy = (_BASE_SIZE - 1) / 2.0
    rows = []
    for s in _SCALES:
        for r in _RATIOS:
            w = _BASE_SIZE * s * math.sqrt(r)
            h = _BASE_SIZE * s / math.sqrt(r)
            rows.append([cx - w / 2.0, cy - h / 2.0, cx + w / 2.0, cy + h / 2.0])
    ba = np.array(rows, dtype=np.float32)
    sx = np.arange(_W, dtype=np.float32) * _STRIDE
    sy = np.arange(_H, dtype=np.float32) * _STRIDE
    SX, SY = np.meshgrid(sx, sy)
    shifts = np.stack([SX.ravel(), SY.ravel(), SX.ravel(), SY.ravel()], axis=1)
    anchors = (ba[None, :, :] + shifts[:, None, :]).reshape(-1, 4)
    aw = anchors[:, 2] - anchors[:, 0] + 1.0
    ah = anchors[:, 3] - anchors[:, 1] + 1.0
    acx = anchors[:, 0] + 0.5 * aw
    acy = anchors[:, 1] + 0.5 * ah
    return np.stack([aw, ah, acx, acy], axis=0).reshape(4, _NR, _NC)


_ANCHOR_PARAMS = _anchor_params()


def _conv1_kernel(x_ref, w_ref, b_ref, o_ref):
    acc = None
    for t in range(9):
        dy, dx = t // 3, t % 3
        win = x_ref[0, dy:dy + 64, dx:dx + 64, :].reshape(_H * _W, 256)
        p = jax.lax.dot_general(win, w_ref[t], (((1,), (0,)), ((), ())),
                                preferred_element_type=jnp.float32)
        acc = p if acc is None else acc + p
    acc = acc + b_ref[0]
    o_ref[0] = jnp.maximum(acc, 0.0).reshape(_H, _W, 256)


def _heads_kernel(x_ref, w_ref, b_ref, o_ref):
    acc = None
    for t in range(9):
        dy, dx = t // 3, t % 3
        win = x_ref[0, dy:dy + 64, dx:dx + 64, :].reshape(_H * _W, 256)
        p = jax.lax.dot_general(win, w_ref[t], (((1,), (0,)), ((), ())),
                                preferred_element_type=jnp.float32)
        acc = p if acc is None else acc + p
    acc = acc + b_ref[0]
    o_ref[0] = acc.reshape(_H, _W, 128)


def _decode_kernel(a_ref, d_ref, o_ref):
    aw, ah, acx, acy = a_ref[0], a_ref[1], a_ref[2], a_ref[3]
    dx, dy, dw, dh, sc = (d_ref[0, i] for i in range(5))
    pcx = dx * aw + acx
    pcy = dy * ah + acy
    pw = jnp.exp(dw) * aw
    ph = jnp.exp(dh) * ah
    x1 = jnp.clip(pcx - 0.5 * pw, 0.0, _IM_W - 1.0)
    y1 = jnp.clip(pcy - 0.5 * ph, 0.0, _IM_H - 1.0)
    x2 = jnp.clip(pcx + 0.5 * pw, 0.0, _IM_W - 1.0)
    y2 = jnp.clip(pcy + 0.5 * ph, 0.0, _IM_H - 1.0)
    w = x2 - x1 + 1.0
    h = y2 - y1 + 1.0
    valid = jnp.logical_and(w >= _MIN_SIZE * _SCALE_F, h >= _MIN_SIZE * _SCALE_F)
    keys = jnp.where(valid, -sc, jnp.inf)
    nv = jnp.sum(valid.astype(jnp.float32))
    o_ref[0, 0] = x1
    o_ref[0, 1] = y1
    o_ref[0, 2] = x2
    o_ref[0, 3] = y2
    o_ref[0, 4] = keys
    o_ref[0, 5] = jnp.full((_NR, _NC), nv, dtype=jnp.float32)


def _nms_kernel(b_ref, o_ref):
    x1, y1, x2, y2, ck = (b_ref[0, i] for i in range(5))
    areas = (x2 - x1 + 1.0) * (y2 - y1 + 1.0)
    rows = jax.lax.broadcasted_iota(jnp.int32, (8, 256), 0)
    cols = jax.lax.broadcasted_iota(jnp.int32, (8, 256), 1)
    idx = rows * 256 + cols
    supp0 = 1.0 - ck
    keep0 = jnp.zeros((8, 256), jnp.float32)

    def body(i, carry):
        supp, keep = carry
        one = idx == i
        alive = 1.0 - jnp.sum(jnp.where(one, supp, 0.0))
        x1i = jnp.sum(jnp.where(one, x1, 0.0))
        y1i = jnp.sum(jnp.where(one, y1, 0.0))
        x2i = jnp.sum(jnp.where(one, x2, 0.0))
        y2i = jnp.sum(jnp.where(one, y2, 0.0))
        ai = jnp.sum(jnp.where(one, areas, 0.0))
        keep = jnp.where(one, alive, keep)
        xx1 = jnp.maximum(x1i, x1)
        yy1 = jnp.maximum(y1i, y1)
        xx2 = jnp.minimum(x2i, x2)
        yy2 = jnp.minimum(y2i, y2)
        inter = jnp.maximum(0.0, xx2 - xx1 + 1.0) * jnp.maximum(0.0, yy2 - yy1 + 1.0)
        iou = inter / (ai + areas - inter)
        hit = jnp.logical_and(iou > _NMS_THRESH, idx > i).astype(jnp.float32)
        supp = jnp.maximum(supp, alive * hit)
        return supp, keep

    _, keep = jax.lax.fori_loop(0, _PRE_NMS, body, (supp0, keep0))
    o_ref[0, 0] = keep


def _finalize_one(keep_f, boxes_top):
    keep = keep_f > 0.5
    nk = jnp.sum(keep.astype(jnp.int32))
    pos = jnp.cumsum(keep.astype(jnp.int32)) - 1
    tgt = jnp.where(keep, pos, _PRE_NMS)
    compact = jnp.zeros((_PRE_NMS,), jnp.int32).at[tgt].set(
        jnp.arange(_PRE_NMS, dtype=jnp.int32), mode='drop')
    denom = jnp.maximum(nk, 1)
    kp = compact[jnp.arange(_POST_NMS, dtype=jnp.int32) % denom]
    g = jnp.take(boxes_top, kp, axis=0)
    return jnp.where(nk == 0, jnp.zeros_like(g), g)


def kernel(feature, conv_w, conv_b, cls_w, cls_b, reg_w, reg_b):
    bsz = feature.shape[0]

    xp = jnp.pad(jnp.transpose(feature, (0, 2, 3, 1)),
                 ((0, 0), (1, 1), (1, 1), (0, 0)))
    w1 = jnp.transpose(conv_w, (2, 3, 1, 0)).reshape(9, 256, 256)
    b1 = jnp.broadcast_to(conv_b[None, :], (8, 256))
    x = pl.pallas_call(
        _conv1_kernel,
        grid=(bsz,),
        in_specs=[
            pl.BlockSpec((1, _H + 2, _W + 2, 256), lambda b: (b, 0, 0, 0)),
            pl.BlockSpec((9, 256, 256), lambda b: (0, 0, 0)),
            pl.BlockSpec((8, 256), lambda b: (0, 0)),
        ],
        out_specs=pl.BlockSpec((1, _H, _W, 256), lambda b: (b, 0, 0, 0)),
        out_shape=jax.ShapeDtypeStruct((bsz, _H, _W, 256), jnp.float32),
    )(xp, w1, b1)

    xq = jnp.pad(x, ((0, 0), (1, 1), (1, 1), (0, 0)))
    w2 = jnp.concatenate([
        jnp.transpose(cls_w, (2, 3, 1, 0)),
        jnp.transpose(reg_w, (2, 3, 1, 0)),
    ], axis=3).reshape(9, 256, 54)
    w2 = jnp.pad(w2, ((0, 0), (0, 0), (0, 128 - 54)))
    b2 = jnp.pad(jnp.concatenate([cls_b, reg_b]), (0, 128 - 54))
    b2 = jnp.broadcast_to(b2[None, :], (8, 128))
    heads = pl.pallas_call(
        _heads_kernel,
        grid=(bsz,),
        in_specs=[
            pl.BlockSpec((1, _H + 2, _W + 2, 256), lambda b: (b, 0, 0, 0)),
            pl.BlockSpec((9, 256, 128), lambda b: (0, 0, 0)),
            pl.BlockSpec((8, 128), lambda b: (0, 0)),
        ],
        out_specs=pl.BlockSpec((1, _H, _W, 128), lambda b: (b, 0, 0, 0)),
        out_shape=jax.ShapeDtypeStruct((bsz, _H, _W, 128), jnp.float32),
    )(xq, w2, b2)

    fg = heads[..., :18].reshape(bsz, _N, 2)[:, :, 1].reshape(bsz, 1, _N)
    deltas = jnp.transpose(heads[..., 18:54].reshape(bsz, _N, 4), (0, 2, 1))
    din = jnp.concatenate([deltas, fg], axis=1).reshape(bsz, 5, _NR, _NC)
    aparams = jnp.asarray(_ANCHOR_PARAMS)

    dec = pl.pallas_call(
        _decode_kernel,
        grid=(bsz,),
        in_specs=[
            pl.BlockSpec((4, _NR, _NC), lambda b: (0, 0, 0)),
            pl.BlockSpec((1, 5, _NR, _NC), lambda b: (b, 0, 0, 0)),
        ],
        out_specs=pl.BlockSpec((1, 6, _NR, _NC), lambda b: (b, 0, 0, 0)),
        out_shape=jax.ShapeDtypeStruct((bsz, 6, _NR, _NC), jnp.float32),
    )(aparams, din)

    prop = jnp.transpose(dec[:, 0:4].reshape(bsz, 4, _N), (0, 2, 1))
    keys = dec[:, 4].reshape(bsz, _N)
    nv = dec[:, 5, 0, 0]

    _, order = jax.lax.top_k(-keys, _PRE_NMS)
    boxes_top = jnp.take_along_axis(prop, order[..., None], axis=1)
    candok = (jnp.arange(_PRE_NMS, dtype=jnp.float32)[None, :] < nv[:, None]
              ).astype(jnp.float32)

    bt = jnp.transpose(boxes_top, (0, 2, 1))
    bt = jnp.pad(bt, ((0, 0), (0, 0), (0, _PAD_N - _PRE_NMS)))
    ck = jnp.pad(candok, ((0, 0), (0, _PAD_N - _PRE_NMS)))[:, None, :]
    bin_ = jnp.concatenate([bt, ck], axis=1).reshape(bsz, 5, 8, 256)

    keep = pl.pallas_call(
        _nms_kernel,
        grid=(bsz,),
        in_specs=[pl.BlockSpec((1, 5, 8, 256), lambda b: (b, 0, 0, 0))],
        out_specs=pl.BlockSpec((1, 1, 8, 256), lambda b: (b, 0, 0, 0)),
        out_shape=jax.ShapeDtypeStruct((bsz, 1, 8, 256), jnp.float32),
    )(bin_)
    keep = keep.reshape(bsz, _PAD_N)[:, :_PRE_NMS]

    return jax.vmap(_finalize_one)(keep, boxes_top)
